# baseline (speedup 1.0000x reference)
import jax
import jax.numpy as jnp
from jax.experimental import pallas as pl
from jax.experimental.pallas import tpu as pltpu

_H = 64
_W = 64
_NT = 32
_W2 = 34
_TROWS = _NT * _W2
_PROWS = _W2 * _W2
_CIN = 256
_CMID = 512


def _rpn_body(x_ref, u_ref, bs_ref, wb_ref, bb_ref, out_ref):
    d = {}
    for r in range(4):
        for c in range(4):
            p, a = r % 2, r // 2
            q, b = c % 2, c // 2
            d[(r, c)] = x_ref[0, p * 2 + q, pl.ds(a * _W2 + b, _TROWS), :]
    e = {}
    for c in range(4):
        e[(0, c)] = d[(0, c)] - d[(2, c)]
        e[(1, c)] = d[(1, c)] + d[(2, c)]
        e[(2, c)] = d[(2, c)] - d[(1, c)]
        e[(3, c)] = d[(1, c)] - d[(3, c)]
    v = {}
    for r in range(4):
        v[(r, 0)] = e[(r, 0)] - e[(r, 2)]
        v[(r, 1)] = e[(r, 1)] + e[(r, 2)]
        v[(r, 2)] = e[(r, 2)] - e[(r, 1)]
        v[(r, 3)] = e[(r, 1)] - e[(r, 3)]
    p0 = [None] * 4
    p1 = [None] * 4
    for r in range(4):
        for c in range(4):
            m = jnp.dot(v[(r, c)], u_ref[r * 4 + c],
                        preferred_element_type=jnp.float32)
            if r == 0:
                p0[c] = m
            elif r == 1:
                p0[c] = p0[c] + m
                p1[c] = m
            elif r == 2:
                p0[c] = p0[c] + m
                p1[c] = p1[c] - m
            else:
                p1[c] = p1[c] - m
    bias = bs_ref[0]
    for s, ps in ((0, p0), (1, p1)):
        q0 = ps[0] + ps[1] + ps[2]
        q1 = ps[1] - ps[2] - ps[3]
        for t, qq in ((0, q0), (1, q1)):
            sh = jnp.maximum(qq + bias, 0.0).astype(jnp.bfloat16)
            z = jnp.dot(sh, wb_ref[...],
                        preferred_element_type=jnp.float32) + bb_ref[0]
            out_ref[0, s * 2 + t, :, 0:18] = z[:, 0:18]
            out_ref[0, s * 2 + t, :, 18:24] = jax.nn.sigmoid(z[:, 18:24])


def kernel(inputs, W_shared, b_shared, W_cls, b_cls, W_reg, b_reg):
    B = inputs.shape[0]
    x_pad = jnp.pad(inputs, ((0, 0), (1, 3), (1, 3), (0, 0)))
    x_planes = (x_pad.reshape(B, _W2, 2, _W2, 2, _CIN)
                .transpose(0, 2, 4, 1, 3, 5)
                .reshape(B, 4, _PROWS, _CIN).astype(jnp.bfloat16))

    g = W_shared
    gt = {}
    for c in range(3):
        gc = g[:, c]
        gt[(0, c)] = gc[0]
        gt[(1, c)] = 0.5 * (gc[0] + gc[1] + gc[2])
        gt[(2, c)] = 0.5 * (gc[0] - gc[1] + gc[2])
        gt[(3, c)] = gc[2]
    u_rows = []
    for r in range(4):
        u_rows.append(gt[(r, 0)])
        u_rows.append(0.5 * (gt[(r, 0)] + gt[(r, 1)] + gt[(r, 2)]))
        u_rows.append(0.5 * (gt[(r, 0)] - gt[(r, 1)] + gt[(r, 2)]))
        u_rows.append(gt[(r, 2)])
    u = jnp.stack(u_rows, axis=0).astype(jnp.bfloat16)

    bs = b_shared.reshape(1, _CMID)
    wc = W_cls.reshape(_CMID, 6)
    wr = W_reg.reshape(_CMID, 12)
    wc3 = wc.reshape(_CMID, 3, 2)
    wdiff = wc3[:, :, 0] - wc3[:, :, 1]
    wd = jnp.stack([wdiff, -wdiff], axis=-1).reshape(_CMID, 6)
    wbig = jnp.concatenate([wc, wr, wd], axis=1).astype(jnp.bfloat16)

    bc3 = b_cls.reshape(3, 2)
    bdiff = bc3[:, 0] - bc3[:, 1]
    bd = jnp.stack([bdiff, -bdiff], axis=-1).reshape(6)
    bbig = jnp.concatenate([b_cls, b_reg, bd]).reshape(1, 24)

    grid_spec = pl.GridSpec(
        grid=(B,),
        in_specs=[
            pl.BlockSpec((1, 4, _PROWS, _CIN), lambda b: (b, 0, 0, 0)),
            pl.BlockSpec((16, _CIN, _CMID), lambda b: (0, 0, 0)),
            pl.BlockSpec((1, _CMID), lambda b: (0, 0)),
            pl.BlockSpec((_CMID, 24), lambda b: (0, 0)),
            pl.BlockSpec((1, 24), lambda b: (0, 0)),
        ],
        out_specs=[
            pl.BlockSpec((1, 4, _TROWS, 24), lambda b: (b, 0, 0, 0)),
        ],
    )
    (out,) = pl.pallas_call(
        _rpn_body,
        grid_spec=grid_spec,
        out_shape=[
            jax.ShapeDtypeStruct((B, 4, _TROWS, 24), jnp.float32),
        ],
        compiler_params=pltpu.CompilerParams(
            dimension_semantics=("arbitrary",),
        ),
    )(x_planes, u, bs, wbig, bbig)

    y = (out.reshape(B, 2, 2, _NT, _W2, 24)[:, :, :, :, :_NT, :]
         .transpose(0, 3, 1, 4, 2, 5)
         .reshape(B, _H * _W, 24))
    n_anch = _H * _W * 3
    rpn_class_logits = y[:, :, 0:6].reshape(B, n_anch, 2)
    rpn_deltas = y[:, :, 6:18].reshape(B, n_anch, 4)
    rpn_probs = y[:, :, 18:24].reshape(B, n_anch, 2)
    return (rpn_class_logits, rpn_probs, rpn_deltas)


# speedup vs baseline: 1.0961x; 1.0961x over previous
import jax
import jax.numpy as jnp
from jax.experimental import pallas as pl
from jax.experimental.pallas import tpu as pltpu

_H = 64
_W = 64
_NT = 32
_NP = 34
_CIN = 256
_CMID = 512


def _rpn_body(x0_ref, x1_ref, u_ref, bs_ref, wb_ref, bb_ref, out_ref):
    planes = {0: x0_ref, 1: x1_ref}
    d = {}
    for r in range(4):
        for c in range(4):
            p, a = r % 2, r // 2
            q, b = c % 2, c // 2
            plane = planes[p]
            d[(r, c)] = plane[0, a:a + _NT, 0, b:b + _NT,
                              q * _CIN:(q + 1) * _CIN].reshape(
                _NT * _NT, _CIN)
    e = {}
    for c in range(4):
        e[(0, c)] = d[(0, c)] - d[(2, c)]
        e[(1, c)] = d[(1, c)] + d[(2, c)]
        e[(2, c)] = d[(2, c)] - d[(1, c)]
        e[(3, c)] = d[(1, c)] - d[(3, c)]
    v = {}
    for r in range(4):
        v[(r, 0)] = e[(r, 0)] - e[(r, 2)]
        v[(r, 1)] = e[(r, 1)] + e[(r, 2)]
        v[(r, 2)] = e[(r, 2)] - e[(r, 1)]
        v[(r, 3)] = e[(r, 1)] - e[(r, 3)]
    p0 = [None] * 4
    p1 = [None] * 4
    for r in range(4):
        for c in range(4):
            m = jnp.dot(v[(r, c)], u_ref[r * 4 + c],
                        preferred_element_type=jnp.float32)
            if r == 0:
                p0[c] = m
            elif r == 1:
                p0[c] = p0[c] + m
                p1[c] = m
            elif r == 2:
                p0[c] = p0[c] + m
                p1[c] = p1[c] - m
            else:
                p1[c] = p1[c] - m
    bias = bs_ref[0]
    for s, ps in ((0, p0), (1, p1)):
        q0 = ps[0] + ps[1] + ps[2]
        q1 = ps[1] - ps[2] - ps[3]
        for t, qq in ((0, q0), (1, q1)):
            sh = jnp.maximum(qq + bias, 0.0).astype(jnp.bfloat16)
            z = jnp.dot(sh, wb_ref[...],
                        preferred_element_type=jnp.float32) + bb_ref[0]
            z3 = jnp.concatenate(
                [z[:, 0:18], jax.nn.sigmoid(z[:, 18:24])], axis=1)
            out_ref[0, s, t, :, :, :] = z3.reshape(_NT, _NT, 24)


def kernel(inputs, W_shared, b_shared, W_cls, b_cls, W_reg, b_reg):
    B = inputs.shape[0]
    x_pad = jnp.pad(inputs, ((0, 0), (1, 3), (1, 3), (0, 0)))
    x7 = x_pad.reshape(B, _NP, 2, _NP, 2 * _CIN).astype(jnp.bfloat16)

    g = W_shared
    gt = {}
    for c in range(3):
        gc = g[:, c]
        gt[(0, c)] = gc[0]
        gt[(1, c)] = 0.5 * (gc[0] + gc[1] + gc[2])
        gt[(2, c)] = 0.5 * (gc[0] - gc[1] + gc[2])
        gt[(3, c)] = gc[2]
    u_rows = []
    for r in range(4):
        u_rows.append(gt[(r, 0)])
        u_rows.append(0.5 * (gt[(r, 0)] + gt[(r, 1)] + gt[(r, 2)]))
        u_rows.append(0.5 * (gt[(r, 0)] - gt[(r, 1)] + gt[(r, 2)]))
        u_rows.append(gt[(r, 2)])
    u = jnp.stack(u_rows, axis=0).astype(jnp.bfloat16)

    bs = b_shared.reshape(1, _CMID)
    wc = W_cls.reshape(_CMID, 6)
    wr = W_reg.reshape(_CMID, 12)
    wc3 = wc.reshape(_CMID, 3, 2)
    wdiff = wc3[:, :, 0] - wc3[:, :, 1]
    wd = jnp.stack([wdiff, -wdiff], axis=-1).reshape(_CMID, 6)
    wbig = jnp.concatenate([wc, wr, wd], axis=1).astype(jnp.bfloat16)

    bc3 = b_cls.reshape(3, 2)
    bdiff = bc3[:, 0] - bc3[:, 1]
    bd = jnp.stack([bdiff, -bdiff], axis=-1).reshape(6)
    bbig = jnp.concatenate([b_cls, b_reg, bd]).reshape(1, 24)

    def plane_spec(p):
        return pl.BlockSpec((1, _NP, 1, _NP, 2 * _CIN),
                            lambda b, p=p: (b, 0, p, 0, 0))

    grid_spec = pl.GridSpec(
        grid=(B,),
        in_specs=[
            plane_spec(0), plane_spec(1),
            pl.BlockSpec((16, _CIN, _CMID), lambda b: (0, 0, 0)),
            pl.BlockSpec((1, _CMID), lambda b: (0, 0)),
            pl.BlockSpec((_CMID, 24), lambda b: (0, 0)),
            pl.BlockSpec((1, 24), lambda b: (0, 0)),
        ],
        out_specs=[
            pl.BlockSpec((1, 2, 2, _NT, _NT, 24), lambda b: (b, 0, 0, 0, 0, 0)),
        ],
    )
    (out,) = pl.pallas_call(
        _rpn_body,
        grid_spec=grid_spec,
        out_shape=[
            jax.ShapeDtypeStruct((B, 2, 2, _NT, _NT, 24), jnp.float32),
        ],
        compiler_params=pltpu.CompilerParams(
            dimension_semantics=("arbitrary",),
        ),
    )(x7, x7, u, bs, wbig, bbig)

    y = (out.transpose(0, 3, 1, 4, 2, 5).reshape(B, _H * _W, 24))
    n_anch = _H * _W * 3
    rpn_class_logits = y[:, :, 0:6].reshape(B, n_anch, 2)
    rpn_deltas = y[:, :, 6:18].reshape(B, n_anch, 4)
    rpn_probs = y[:, :, 18:24].reshape(B, n_anch, 2)
    return (rpn_class_logits, rpn_probs, rpn_deltas)


# speedup vs baseline: 1.2786x; 1.1665x over previous
import jax
import jax.numpy as jnp
import numpy as np
from jax.experimental import pallas as pl
from jax.experimental.pallas import tpu as pltpu

_H = 64
_W = 64
_NT = 32
_NP = 34
_CIN = 256
_CMID = 512


def _rpn_body(x0_ref, x1_ref, u_ref, bs_ref, wb_ref, bb_ref, out_ref):
    planes = {0: x0_ref, 1: x1_ref}
    d = {}
    for r in range(4):
        for c in range(4):
            p, a = r % 2, r // 2
            q, b = c % 2, c // 2
            plane = planes[p]
            d[(r, c)] = plane[0, a:a + _NT, 0, b:b + _NT,
                              q * _CIN:(q + 1) * _CIN].reshape(
                _NT * _NT, _CIN)
    e = {}
    for c in range(4):
        e[(0, c)] = d[(0, c)] - d[(2, c)]
        e[(1, c)] = d[(1, c)] + d[(2, c)]
        e[(2, c)] = d[(2, c)] - d[(1, c)]
        e[(3, c)] = d[(1, c)] - d[(3, c)]
    v = {}
    for r in range(4):
        v[(r, 0)] = e[(r, 0)] - e[(r, 2)]
        v[(r, 1)] = e[(r, 1)] + e[(r, 2)]
        v[(r, 2)] = e[(r, 2)] - e[(r, 1)]
        v[(r, 3)] = e[(r, 1)] - e[(r, 3)]
    p0 = [None] * 4
    p1 = [None] * 4
    for r in range(4):
        for c in range(4):
            m = jnp.dot(v[(r, c)], u_ref[r * 4 + c],
                        preferred_element_type=jnp.float32)
            if r == 0:
                p0[c] = m
            elif r == 1:
                p0[c] = p0[c] + m
                p1[c] = m
            elif r == 2:
                p0[c] = p0[c] + m
                p1[c] = p1[c] - m
            else:
                p1[c] = p1[c] - m
    bias = bs_ref[0]
    for s, ps in ((0, p0), (1, p1)):
        q0 = ps[0] + ps[1] + ps[2]
        q1 = ps[1] - ps[2] - ps[3]
        zt = []
        for qq in (q0, q1):
            sh = jnp.maximum(qq + bias, 0.0).astype(jnp.bfloat16)
            z = jnp.dot(sh, wb_ref[...],
                        preferred_element_type=jnp.float32) + bb_ref[0]
            z3 = jnp.concatenate(
                [z[:, 0:18], jax.nn.sigmoid(z[:, 18:24])], axis=1)
            zt.append(z3.reshape(_NT, _NT, 24))
        out_ref[0, :, s, :, :] = jnp.concatenate(zt, axis=2)


def kernel(inputs, W_shared, b_shared, W_cls, b_cls, W_reg, b_reg):
    B = inputs.shape[0]
    x_pad = jnp.pad(inputs, ((0, 0), (1, 3), (1, 3), (0, 0)))
    x7 = x_pad.reshape(B, _NP, 2, _NP, 2 * _CIN).astype(jnp.bfloat16)

    gmat = np.array([[1.0, 0.0, 0.0],
                     [0.5, 0.5, 0.5],
                     [0.5, -0.5, 0.5],
                     [0.0, 0.0, 1.0]], dtype=np.float32)
    tmat = np.einsum('rk,cl->rckl', gmat, gmat).reshape(16, 9)
    u = (jnp.dot(jnp.asarray(tmat), W_shared.reshape(9, _CIN * _CMID))
         .reshape(16, _CIN, _CMID).astype(jnp.bfloat16))

    bs = b_shared.reshape(1, _CMID)
    wc = W_cls.reshape(_CMID, 6)
    wr = W_reg.reshape(_CMID, 12)
    wc3 = wc.reshape(_CMID, 3, 2)
    wdiff = wc3[:, :, 0] - wc3[:, :, 1]
    wd = jnp.stack([wdiff, -wdiff], axis=-1).reshape(_CMID, 6)
    wbig = jnp.concatenate([wc, wr, wd], axis=1).astype(jnp.bfloat16)

    bc3 = b_cls.reshape(3, 2)
    bdiff = bc3[:, 0] - bc3[:, 1]
    bd = jnp.stack([bdiff, -bdiff], axis=-1).reshape(6)
    bbig = jnp.concatenate([b_cls, b_reg, bd]).reshape(1, 24)

    def plane_spec(p):
        return pl.BlockSpec((1, _NP, 1, _NP, 2 * _CIN),
                            lambda b, p=p: (b, 0, p, 0, 0))

    grid_spec = pl.GridSpec(
        grid=(B,),
        in_specs=[
            plane_spec(0), plane_spec(1),
            pl.BlockSpec((16, _CIN, _CMID), lambda b: (0, 0, 0)),
            pl.BlockSpec((1, _CMID), lambda b: (0, 0)),
            pl.BlockSpec((_CMID, 24), lambda b: (0, 0)),
            pl.BlockSpec((1, 24), lambda b: (0, 0)),
        ],
        out_specs=[
            pl.BlockSpec((1, _NT, 2, _NT, 48), lambda b: (b, 0, 0, 0, 0)),
        ],
    )
    (out,) = pl.pallas_call(
        _rpn_body,
        grid_spec=grid_spec,
        out_shape=[
            jax.ShapeDtypeStruct((B, _NT, 2, _NT, 48), jnp.float32),
        ],
        compiler_params=pltpu.CompilerParams(
            dimension_semantics=("arbitrary",),
        ),
    )(x7, x7, u, bs, wbig, bbig)

    y = out.reshape(B, _H * _W, 24)
    n_anch = _H * _W * 3
    rpn_class_logits = y[:, :, 0:6].reshape(B, n_anch, 2)
    rpn_deltas = y[:, :, 6:18].reshape(B, n_anch, 4)
    rpn_probs = y[:, :, 18:24].reshape(B, n_anch, 2)
    return (rpn_class_logits, rpn_probs, rpn_deltas)


# speedup vs baseline: 1.5922x; 1.2453x over previous
import jax
import jax.numpy as jnp
from jax.experimental import pallas as pl
from jax.experimental.pallas import tpu as pltpu

_H = 64
_W = 64
_NT = 32
_CIN = 256
_CMID = 512
_G = ((1.0, 0.0, 0.0),
      (0.5, 0.5, 0.5),
      (0.5, -0.5, 0.5),
      (0.0, 0.0, 1.0))


def _rpn_body(x0_ref, x1_ref, g_ref, bs_ref, wb_ref, bb_ref,
              lg_ref, pb_ref, dl_ref, u_scr):
    @pl.when(pl.program_id(0) == 0)
    def _():
        slabs = [g_ref[pl.ds(k * _CIN, _CIN), :] for k in range(9)]
        for t in range(16):
            r, c = t // 4, t % 4
            acc = None
            for k in range(9):
                coef = _G[r][k // 3] * _G[c][k % 3]
                if coef == 0.0:
                    continue
                term = slabs[k] if coef == 1.0 else slabs[k] * coef
                acc = term if acc is None else acc + term
            u_scr[t] = acc.astype(jnp.bfloat16)

    xq = {}
    for p, ref in ((0, x0_ref), (1, x1_ref)):
        plane = ref[0, :, 0, :, :].astype(jnp.bfloat16)
        for q in range(2):
            xq[(p, q)] = plane[:, :, q * _CIN:(q + 1) * _CIN]
    zrow = jnp.zeros((1, _NT, _CIN), jnp.bfloat16)
    zcol = jnp.zeros((_NT, 1, _CIN), jnp.bfloat16)

    def dslice(r, c):
        P, a = r % 2, r // 2
        Q, b = c % 2, c // 2
        base = xq[(1 - P, 1 - Q)]
        if P == 0 and a == 0:
            base = jnp.concatenate([zrow, base[0:_NT - 1]], axis=0)
        elif P == 1 and a == 1:
            base = jnp.concatenate([base[1:_NT], zrow], axis=0)
        if Q == 0 and b == 0:
            base = jnp.concatenate([zcol, base[:, 0:_NT - 1]], axis=1)
        elif Q == 1 and b == 1:
            base = jnp.concatenate([base[:, 1:_NT], zcol], axis=1)
        return base.reshape(_NT * _NT, _CIN)

    d = {(r, c): dslice(r, c) for r in range(4) for c in range(4)}
    e = {}
    for c in range(4):
        e[(0, c)] = d[(0, c)] - d[(2, c)]
        e[(1, c)] = d[(1, c)] + d[(2, c)]
        e[(2, c)] = d[(2, c)] - d[(1, c)]
        e[(3, c)] = d[(1, c)] - d[(3, c)]
    v = {}
    for r in range(4):
        v[(r, 0)] = e[(r, 0)] - e[(r, 2)]
        v[(r, 1)] = e[(r, 1)] + e[(r, 2)]
        v[(r, 2)] = e[(r, 2)] - e[(r, 1)]
        v[(r, 3)] = e[(r, 1)] - e[(r, 3)]
    p0 = [None] * 4
    p1 = [None] * 4
    for r in range(4):
        for c in range(4):
            m = jnp.dot(v[(r, c)], u_scr[r * 4 + c],
                        preferred_element_type=jnp.float32)
            if r == 0:
                p0[c] = m
            elif r == 1:
                p0[c] = p0[c] + m
                p1[c] = m
            elif r == 2:
                p0[c] = p0[c] + m
                p1[c] = p1[c] - m
            else:
                p1[c] = p1[c] - m
    bias = bs_ref[0]
    for s, ps in ((0, p0), (1, p1)):
        q0 = ps[0] + ps[1] + ps[2]
        q1 = ps[1] - ps[2] - ps[3]
        lg, pb, dl = [], [], []
        for qq in (q0, q1):
            sh = jnp.maximum(qq + bias, 0.0).astype(jnp.bfloat16)
            z = jnp.dot(sh, wb_ref[...],
                        preferred_element_type=jnp.float32) + bb_ref[0]
            lg.append(z[:, 0:6].reshape(_NT, _NT, 6))
            dl.append(z[:, 6:18].reshape(_NT, _NT, 12))
            pb.append(jax.nn.sigmoid(z[:, 18:24]).reshape(_NT, _NT, 6))
        lg_ref[0, :, s, :, :] = jnp.concatenate(lg, axis=2)
        pb_ref[0, :, s, :, :] = jnp.concatenate(pb, axis=2)
        dl_ref[0, :, s, :, :] = jnp.concatenate(dl, axis=2)


def kernel(inputs, W_shared, b_shared, W_cls, b_cls, W_reg, b_reg):
    B = inputs.shape[0]
    x8 = inputs.reshape(B, _NT, 2, _NT, 2 * _CIN)
    g2 = W_shared.reshape(9 * _CIN, _CMID)

    bs = b_shared.reshape(1, _CMID)
    wc = W_cls.reshape(_CMID, 6)
    wr = W_reg.reshape(_CMID, 12)
    wc3 = wc.reshape(_CMID, 3, 2)
    wdiff = wc3[:, :, 0] - wc3[:, :, 1]
    wd = jnp.stack([wdiff, -wdiff], axis=-1).reshape(_CMID, 6)
    wbig = jnp.concatenate([wc, wr, wd], axis=1).astype(jnp.bfloat16)

    bc3 = b_cls.reshape(3, 2)
    bdiff = bc3[:, 0] - bc3[:, 1]
    bd = jnp.stack([bdiff, -bdiff], axis=-1).reshape(6)
    bbig = jnp.concatenate([b_cls, b_reg, bd]).reshape(1, 24)

    def plane_spec(p):
        return pl.BlockSpec((1, _NT, 1, _NT, 2 * _CIN),
                            lambda b, p=p: (b, 0, p, 0, 0))

    in_specs = [
            plane_spec(0), plane_spec(1),
            pl.BlockSpec((9 * _CIN, _CMID), lambda b: (0, 0)),
            pl.BlockSpec((1, _CMID), lambda b: (0, 0)),
            pl.BlockSpec((_CMID, 24), lambda b: (0, 0)),
            pl.BlockSpec((1, 24), lambda b: (0, 0)),
    ]
    out_specs = [
            pl.BlockSpec((1, _NT, 2, _NT, 12), lambda b: (b, 0, 0, 0, 0)),
            pl.BlockSpec((1, _NT, 2, _NT, 12), lambda b: (b, 0, 0, 0, 0)),
            pl.BlockSpec((1, _NT, 2, _NT, 24), lambda b: (b, 0, 0, 0, 0)),
    ]
    lg, pb, dl = pl.pallas_call(
        _rpn_body,
        grid=(B,),
        in_specs=in_specs,
        out_specs=out_specs,
        out_shape=[
            jax.ShapeDtypeStruct((B, _NT, 2, _NT, 12), jnp.float32),
            jax.ShapeDtypeStruct((B, _NT, 2, _NT, 12), jnp.float32),
            jax.ShapeDtypeStruct((B, _NT, 2, _NT, 24), jnp.float32),
        ],
        scratch_shapes=[pltpu.VMEM((16, _CIN, _CMID), jnp.bfloat16)],
        compiler_params=pltpu.CompilerParams(
            dimension_semantics=("arbitrary",),
        ),
    )(x8, x8, g2, bs, wbig, bbig)

    n_anch = _H * _W * 3
    rpn_class_logits = lg.reshape(B, n_anch, 2)
    rpn_probs = pb.reshape(B, n_anch, 2)
    rpn_deltas = dl.reshape(B, n_anch, 4)
    return (rpn_class_logits, rpn_probs, rpn_deltas)
